# baseline (speedup 1.0000x reference)
import functools

import jax
import jax.numpy as jnp
from jax import lax
from jax.experimental import pallas as pl
from jax.experimental.pallas import tpu as pltpu
from jax.experimental.pallas import tpu_sc as plsc

VOCAB = 100000
MAX_SEQ = 200
EMBED = 64
BATCH = 4096

NROWS = BATCH * MAX_SEQ
_INFO = plsc.get_sparse_core_info()
NC, NS, L = _INFO.num_cores, _INFO.num_subcores, _INFO.num_lanes
NW = NC * NS
ROWS_PER_W = NROWS // NW
SEQ_PER_CHUNK = 4
CHUNK = SEQ_PER_CHUNK * MAX_SEQ
HCHUNK = CHUNK // 2
NCHUNKS = ROWS_PER_W // CHUNK
NPAIRS = NCHUNKS // 2
SUBG = 80
NSUBG = CHUNK // SUBG
D_SLICES = EMBED // L


def _emb_body(x_hbm, tok_hbm, pos_hbm, out_hbm,
              idx0, idx1, rows0, rows1, pos_v, gsem0, gsem1, ssem0, ssem1):
    wid = lax.axis_index("s") * NC + lax.axis_index("c")
    wbase2 = wid * (ROWS_PER_W // 2)

    pltpu.sync_copy(pos_hbm, pos_v)

    def stage_idx(ci, idx_v):
        base2 = wbase2 + ci * HCHUNK
        pltpu.sync_copy(x_hbm.at[pl.ds(base2, HCHUNK)],
                        idx_v.at[pl.ds(0, HCHUNK)])
        pltpu.sync_copy(x_hbm.at[pl.ds(NROWS // 2 + base2, HCHUNK)],
                        idx_v.at[pl.ds(HCHUNK, HCHUNK)])

    def fire_gathers(idx_v, rows_v, sem):
        for g in range(NSUBG):
            pltpu.async_copy(
                tok_hbm.at[idx_v.at[pl.ds(g * SUBG, SUBG)]],
                rows_v.at[pl.ds(g * SUBG, SUBG)], sem)

    def drain_gather(sem, rows_v):
        pltpu.make_async_copy(
            tok_hbm.at[pl.ds(0, CHUNK)], rows_v, sem).wait()

    def store(rows_v, ci, sem):
        base2 = wbase2 + ci * HCHUNK
        pltpu.async_copy(rows_v.at[pl.ds(0, HCHUNK)],
                         out_hbm.at[pl.ds(base2, HCHUNK), pl.ds(0, EMBED)],
                         sem)
        pltpu.async_copy(rows_v.at[pl.ds(HCHUNK, HCHUNK)],
                         out_hbm.at[pl.ds(base2, HCHUNK),
                                    pl.ds(EMBED, EMBED)],
                         sem)

    def drain_store(sem, rows_v):
        for h in range(2):
            pltpu.make_async_copy(
                out_hbm.at[pl.ds(0, HCHUNK), pl.ds(0, EMBED)],
                rows_v.at[pl.ds(h * HCHUNK, HCHUNK)], sem).wait()

    def add_pos(rows_v):
        def add_body(s2, carry):
            for par in range(2):
                s = 2 * s2 + par
                for c in range(D_SLICES):
                    p = pos_v[s, pl.ds(c * L, L)]
                    for q in range(SEQ_PER_CHUNK):
                        r = par * HCHUNK + q * (MAX_SEQ // 2) + s2
                        rows_v[r, pl.ds(c * L, L)] = (
                            rows_v[r, pl.ds(c * L, L)] + p)
            return carry
        lax.fori_loop(0, MAX_SEQ // 2, add_body, None)

    stage_idx(0, idx0)
    fire_gathers(idx0, rows0, gsem0)

    def pair_body(ci2, _):
        c0 = 2 * ci2

        stage_idx(c0 + 1, idx1)

        @pl.when(ci2 > 0)
        def _():
            drain_store(ssem1, rows1)

        fire_gathers(idx1, rows1, gsem1)

        drain_gather(gsem0, rows0)
        add_pos(rows0)
        store(rows0, c0, ssem0)

        @pl.when(ci2 + 1 < NPAIRS)
        def _():
            stage_idx(c0 + 2, idx0)
            drain_store(ssem0, rows0)
            fire_gathers(idx0, rows0, gsem0)

        drain_gather(gsem1, rows1)
        add_pos(rows1)
        store(rows1, c0 + 1, ssem1)
        return _

    lax.fori_loop(0, NPAIRS, pair_body, None)
    drain_store(ssem0, rows0)
    drain_store(ssem1, rows1)


@functools.partial(
    pl.kernel,
    mesh=plsc.VectorSubcoreMesh(core_axis_name="c", subcore_axis_name="s"),
    compiler_params=pltpu.CompilerParams(use_tc_tiling_on_sc=False),
    out_type=jax.ShapeDtypeStruct((NROWS // 2, 2 * EMBED), jnp.float32),
    scratch_types=[
        pltpu.VMEM((CHUNK,), jnp.int32),
        pltpu.VMEM((CHUNK,), jnp.int32),
        pltpu.VMEM((CHUNK, EMBED), jnp.float32),
        pltpu.VMEM((CHUNK, EMBED), jnp.float32),
        pltpu.VMEM((MAX_SEQ, EMBED), jnp.float32),
        pltpu.SemaphoreType.DMA,
        pltpu.SemaphoreType.DMA,
        pltpu.SemaphoreType.DMA,
        pltpu.SemaphoreType.DMA,
    ],
)
def _emb_kernel(x_hbm, tok_hbm, pos_hbm, out_hbm,
                idx0, idx1, rows0, rows1, pos_v, gsem0, gsem1, ssem0, ssem1):
    _emb_body(x_hbm, tok_hbm, pos_hbm, out_hbm,
              idx0, idx1, rows0, rows1, pos_v, gsem0, gsem1, ssem0, ssem1)


def kernel(x, token_table, pos_table):
    x_flat = x.reshape(-1).astype(jnp.int32)
    pairs = x_flat.reshape(-1, 2)
    x_eo = jnp.concatenate([pairs[:, 0], pairs[:, 1]])
    out = _emb_kernel(x_eo, token_table, pos_table)
    return out.reshape(BATCH, MAX_SEQ, EMBED)


# speedup vs baseline: 1.2279x; 1.2279x over previous
import functools

import jax
import jax.numpy as jnp
from jax import lax
from jax.experimental import pallas as pl
from jax.experimental.pallas import tpu as pltpu
from jax.experimental.pallas import tpu_sc as plsc

VOCAB = 100000
MAX_SEQ = 200
EMBED = 64
BATCH = 4096
PADD = 2 * EMBED

NROWS = BATCH * MAX_SEQ
_INFO = plsc.get_sparse_core_info()
NC, NS, L = _INFO.num_cores, _INFO.num_subcores, _INFO.num_lanes
NW = NC * NS
ROWS_PER_W = NROWS // NW
CHUNK = MAX_SEQ
NCHUNKS = ROWS_PER_W // CHUNK
NPAIRS = NCHUNKS // 2
SUBG = 40
NSUBG = CHUNK // SUBG
D_SLICES = EMBED // L


def _emb_body(x_hbm, tok_hbm, pos_hbm, out_hbm,
              idx0, idx1, eo0, eo1, rows0, rows1, pos_v,
              gsem0, gsem1, ssem0, ssem1):
    wid = lax.axis_index("s") * NC + lax.axis_index("c")
    wbase = wid * ROWS_PER_W

    pltpu.sync_copy(pos_hbm, pos_v)

    def fire_gathers(idx_v, eo_v, sem):
        for g in range(NSUBG):
            pltpu.async_copy(
                tok_hbm.at[idx_v.at[pl.ds(g * SUBG, SUBG)]],
                eo_v.at[pl.ds(g * SUBG, SUBG)], sem)

    def drain_gather(sem, eo_v):
        pltpu.make_async_copy(
            tok_hbm.at[pl.ds(0, CHUNK)], eo_v, sem).wait()

    def drain_store(sem, rows_v):
        pltpu.make_async_copy(
            out_hbm.at[pl.ds(0, CHUNK)], rows_v, sem).wait()

    def add_pos(eo_v, rows_v):
        def add_body(s, carry):
            for c in range(D_SLICES):
                p = pos_v[s, pl.ds(c * L, L)]
                rows_v[s, pl.ds(c * L, L)] = eo_v[s, pl.ds(c * L, L)] + p
            return carry
        lax.fori_loop(0, MAX_SEQ, add_body, None)

    pltpu.sync_copy(x_hbm.at[pl.ds(wbase, CHUNK)], idx0)
    fire_gathers(idx0, eo0, gsem0)

    def pair_body(ci2, _):
        c0 = 2 * ci2
        base0 = wbase + c0 * CHUNK
        base1 = base0 + CHUNK

        pltpu.sync_copy(x_hbm.at[pl.ds(base1, CHUNK)], idx1)
        fire_gathers(idx1, eo1, gsem1)

        drain_gather(gsem0, eo0)

        @pl.when(ci2 > 0)
        def _():
            drain_store(ssem0, rows0)

        add_pos(eo0, rows0)
        pltpu.async_copy(rows0, out_hbm.at[pl.ds(base0, CHUNK)], ssem0)

        @pl.when(ci2 + 1 < NPAIRS)
        def _():
            pltpu.sync_copy(x_hbm.at[pl.ds(base1 + CHUNK, CHUNK)], idx0)
            fire_gathers(idx0, eo0, gsem0)

        drain_gather(gsem1, eo1)

        @pl.when(ci2 > 0)
        def _():
            drain_store(ssem1, rows1)

        add_pos(eo1, rows1)
        pltpu.async_copy(rows1, out_hbm.at[pl.ds(base1, CHUNK)], ssem1)
        return _

    lax.fori_loop(0, NPAIRS, pair_body, None)
    drain_store(ssem0, rows0)
    drain_store(ssem1, rows1)


@functools.partial(
    pl.kernel,
    mesh=plsc.VectorSubcoreMesh(core_axis_name="c", subcore_axis_name="s"),
    compiler_params=pltpu.CompilerParams(use_tc_tiling_on_sc=False),
    out_type=jax.ShapeDtypeStruct((NROWS, PADD), jnp.float32),
    scratch_types=[
        pltpu.VMEM((CHUNK,), jnp.int32),
        pltpu.VMEM((CHUNK,), jnp.int32),
        pltpu.VMEM((CHUNK, EMBED), jnp.float32),
        pltpu.VMEM((CHUNK, EMBED), jnp.float32),
        pltpu.VMEM((CHUNK, PADD), jnp.float32),
        pltpu.VMEM((CHUNK, PADD), jnp.float32),
        pltpu.VMEM((MAX_SEQ, EMBED), jnp.float32),
        pltpu.SemaphoreType.DMA,
        pltpu.SemaphoreType.DMA,
        pltpu.SemaphoreType.DMA,
        pltpu.SemaphoreType.DMA,
    ],
)
def _emb_kernel(x_hbm, tok_hbm, pos_hbm, out_hbm,
                idx0, idx1, eo0, eo1, rows0, rows1, pos_v,
                gsem0, gsem1, ssem0, ssem1):
    _emb_body(x_hbm, tok_hbm, pos_hbm, out_hbm,
              idx0, idx1, eo0, eo1, rows0, rows1, pos_v,
              gsem0, gsem1, ssem0, ssem1)


def kernel(x, token_table, pos_table):
    x_flat = x.reshape(-1).astype(jnp.int32)
    out = _emb_kernel(x_flat, token_table, pos_table)
    return out[:, :EMBED].reshape(BATCH, MAX_SEQ, EMBED)


# speedup vs baseline: 1.6521x; 1.3454x over previous
import functools

import jax
import jax.numpy as jnp
from jax import lax
from jax.experimental import pallas as pl
from jax.experimental.pallas import tpu as pltpu
from jax.experimental.pallas import tpu_sc as plsc

VOCAB = 100000
MAX_SEQ = 200
EMBED = 64
BATCH = 4096
PADD = 2 * EMBED

NROWS = BATCH * MAX_SEQ
_INFO = plsc.get_sparse_core_info()
NC, NS, L = _INFO.num_cores, _INFO.num_subcores, _INFO.num_lanes
NW = NC * NS
ROWS_PER_W = NROWS // NW
SEQ_PER_CHUNK = 2
CHUNK = SEQ_PER_CHUNK * MAX_SEQ
NCHUNKS = ROWS_PER_W // CHUNK
NPAIRS = NCHUNKS // 2
SUBG = 80
NSUBG = CHUNK // SUBG
D_SLICES = EMBED // L


def _emb_body(x_hbm, tok_hbm, pos_hbm, out_hbm,
              idx0, idx1, rows0, rows1, pos_v, gsem0, gsem1, ssem0, ssem1):
    wid = lax.axis_index("s") * NC + lax.axis_index("c")
    wbase = wid * ROWS_PER_W

    pltpu.sync_copy(pos_hbm, pos_v)

    def fire_gathers(idx_v, rows_v, sem):
        for g in range(NSUBG):
            pltpu.async_copy(
                tok_hbm.at[idx_v.at[pl.ds(g * SUBG, SUBG)]],
                rows_v.at[pl.ds(g * SUBG, SUBG)], sem)

    def drain(sem, rows_v):
        pltpu.make_async_copy(out_hbm.at[pl.ds(0, CHUNK)], rows_v, sem).wait()

    def add_pos(rows_v):
        def add_body(s, carry):
            for c in range(D_SLICES):
                p = pos_v[s, pl.ds(c * L, L)]
                for q in range(SEQ_PER_CHUNK):
                    r = q * MAX_SEQ + s
                    rows_v[r, pl.ds(c * L, L)] = rows_v[r, pl.ds(c * L, L)] + p
            return carry
        lax.fori_loop(0, MAX_SEQ, add_body, None)

    pltpu.sync_copy(x_hbm.at[pl.ds(wbase, CHUNK)], idx0)
    fire_gathers(idx0, rows0, gsem0)

    def pair_body(ci2, _):
        c0 = 2 * ci2
        base0 = wbase + c0 * CHUNK
        base1 = base0 + CHUNK

        pltpu.sync_copy(x_hbm.at[pl.ds(base1, CHUNK)], idx1)

        @pl.when(ci2 > 0)
        def _():
            drain(ssem1, rows1)

        fire_gathers(idx1, rows1, gsem1)

        drain(gsem0, rows0)
        add_pos(rows0)
        pltpu.async_copy(rows0, out_hbm.at[pl.ds(base0, CHUNK)], ssem0)

        @pl.when(ci2 + 1 < NPAIRS)
        def _():
            pltpu.sync_copy(x_hbm.at[pl.ds(base1 + CHUNK, CHUNK)], idx0)
            drain(ssem0, rows0)
            fire_gathers(idx0, rows0, gsem0)

        drain(gsem1, rows1)
        add_pos(rows1)
        pltpu.async_copy(rows1, out_hbm.at[pl.ds(base1, CHUNK)], ssem1)
        return _

    lax.fori_loop(0, NPAIRS, pair_body, None)
    drain(ssem0, rows0)
    drain(ssem1, rows1)


@functools.partial(
    pl.kernel,
    mesh=plsc.VectorSubcoreMesh(core_axis_name="c", subcore_axis_name="s"),
    compiler_params=pltpu.CompilerParams(use_tc_tiling_on_sc=False),
    out_type=jax.ShapeDtypeStruct((NROWS, PADD), jnp.float32),
    scratch_types=[
        pltpu.VMEM((CHUNK,), jnp.int32),
        pltpu.VMEM((CHUNK,), jnp.int32),
        pltpu.VMEM((CHUNK, PADD), jnp.float32),
        pltpu.VMEM((CHUNK, PADD), jnp.float32),
        pltpu.VMEM((MAX_SEQ, EMBED), jnp.float32),
        pltpu.SemaphoreType.DMA,
        pltpu.SemaphoreType.DMA,
        pltpu.SemaphoreType.DMA,
        pltpu.SemaphoreType.DMA,
    ],
)
def _emb_kernel(x_hbm, tok_hbm, pos_hbm, out_hbm,
                idx0, idx1, rows0, rows1, pos_v, gsem0, gsem1, ssem0, ssem1):
    _emb_body(x_hbm, tok_hbm, pos_hbm, out_hbm,
              idx0, idx1, rows0, rows1, pos_v, gsem0, gsem1, ssem0, ssem1)


def kernel(x, token_table, pos_table):
    x_flat = x.reshape(-1).astype(jnp.int32)
    tok_pad = jnp.pad(token_table, ((0, 0), (0, PADD - EMBED)))
    out = _emb_kernel(x_flat, tok_pad, pos_table)
    return out[:, :EMBED].reshape(BATCH, MAX_SEQ, EMBED)
